# baseline (speedup 1.0000x reference)
import functools

import jax
import jax.numpy as jnp
from jax import lax
from jax.experimental import pallas as pl
from jax.experimental.pallas import tpu as pltpu
from jax.experimental.pallas import tpu_sc as plsc

_N = 10000
_D = 128
_E = 320000
_EPS = 1e-5

_NC = 2
_NS = 16
_CHUNK = 128
_NBUF = 2
_CH = 80
_EPT = _CH * _CHUNK
_EPAD = _NC * _NS * _EPT
_HALF = _CH // 2
_HPAD = 10016
_NPAD = 10112
_ZRPT = _NPAD // _NS


def _bn_relu(y):
    mean = jnp.mean(y, axis=0, keepdims=True)
    cen = y - mean
    var = jnp.mean(cen * cen, axis=0, keepdims=True)
    return jnp.maximum(cen * lax.rsqrt(var + _EPS), 0.0)


def _fc_in_body(x_ref, w_ref, o_ref):
    y = jnp.dot(x_ref[...], w_ref[...], preferred_element_type=jnp.float32)
    o_ref[pl.ds(0, _N), :] = _bn_relu(y)
    o_ref[pl.ds(_N, _HPAD - _N), :] = jnp.zeros((_HPAD - _N, _D), jnp.float32)


def _fc_mid_body(p_ref, w_ref, b_ref, o_ref):
    a = p_ref[0, pl.ds(0, _N), :] + p_ref[1, pl.ds(0, _N), :]
    y = jnp.dot(a, w_ref[...], preferred_element_type=jnp.float32)
    o_ref[pl.ds(0, _N), :] = _bn_relu(y + b_ref[...])
    o_ref[pl.ds(_N, _HPAD - _N), :] = jnp.zeros((_HPAD - _N, _D), jnp.float32)


def _fc_fin_body(p_ref, w_ref, b_ref, o_ref):
    a = p_ref[0, pl.ds(0, _N), :] + p_ref[1, pl.ds(0, _N), :]
    y = jnp.dot(a, w_ref[...], preferred_element_type=jnp.float32)
    o_ref[...] = _bn_relu(y + b_ref[...])


_dense_in = pl.pallas_call(
    _fc_in_body, out_shape=jax.ShapeDtypeStruct((_HPAD, _D), jnp.float32))
_dense_mid = pl.pallas_call(
    _fc_mid_body, out_shape=jax.ShapeDtypeStruct((_HPAD, _D), jnp.float32))
_dense_fin = pl.pallas_call(
    _fc_fin_body, out_shape=jax.ShapeDtypeStruct((_N, _D), jnp.float32))


@functools.partial(
    pl.kernel,
    mesh=plsc.VectorSubcoreMesh(core_axis_name="c", subcore_axis_name="s"),
    out_type=jax.ShapeDtypeStruct((_NC, _NPAD, _D), jnp.float32),
    scratch_types=[
        pltpu.VMEM((_HALF, _CHUNK), jnp.int32),
        pltpu.VMEM((_HALF, _CHUNK), jnp.int32),
        *[pltpu.VMEM((_CHUNK, _D), jnp.float32) for _ in range(_NBUF)],
        pltpu.VMEM_SHARED((_NPAD, _D), jnp.float32),
        *[pltpu.SemaphoreType.DMA for _ in range(_NBUF)],
    ],
)
def _sc_agg(h_hbm, src_hbm, dst_hbm, z_hbm, out_hbm,
            src_v, dst_v, r0, r1, agg_sh, s0, s1):
    rows = (r0, r1)
    sems = (s0, s1)
    c = lax.axis_index("c")
    s = lax.axis_index("s")
    with jax.named_scope("zero_phase"):
        pltpu.sync_copy(z_hbm, rows[0])
        for k in range(4):
            pltpu.sync_copy(rows[0],
                            agg_sh.at[pl.ds(s * _ZRPT + k * _CHUNK, _CHUNK)])
        pltpu.sync_copy(rows[0].at[pl.ds(0, _ZRPT - 4 * _CHUNK)],
                        agg_sh.at[pl.ds(s * _ZRPT + 4 * _CHUNK,
                                        _ZRPT - 4 * _CHUNK)])
        plsc.subcore_barrier()
    for half in range(2):
        with jax.named_scope(f"edge_half{half}"):
            pltpu.sync_copy(src_hbm.at[c, s, half], src_v)
            pltpu.sync_copy(dst_hbm.at[c, s, half], dst_v)
            for b in range(_NBUF):
                pltpu.async_copy(h_hbm.at[src_v.at[b]], rows[b], sems[b])

            def body(i, carry):
                for b in range(_NBUF):
                    j = i * _NBUF + b
                    pltpu.make_async_copy(h_hbm.at[src_v.at[j]], rows[b],
                                          sems[b]).wait()
                    pltpu.sync_copy(rows[b], agg_sh.at[dst_v.at[j]], add=True)
                    nxt = j + _NBUF

                    @pl.when(nxt < _HALF)
                    def _():
                        pltpu.async_copy(h_hbm.at[src_v.at[nxt]], rows[b],
                                         sems[b])

                return carry

            lax.fori_loop(0, _HALF // _NBUF, body, 0)
    with jax.named_scope("writeout"):
        plsc.subcore_barrier()
        pltpu.sync_copy(agg_sh.at[pl.ds(s * _ZRPT, _ZRPT)],
                        out_hbm.at[c, pl.ds(s * _ZRPT, _ZRPT)])


def kernel(x, edge_index, W_init, W1, b1, W2, b2):
    src = edge_index[0]
    dst = edge_index[1]
    pad = _EPAD - _E
    src_p = jnp.concatenate(
        [src, jnp.full((pad,), _N, jnp.int32)]).reshape(
            _NC, _NS, 2, _HALF, _CHUNK)
    dst_p = jnp.concatenate(
        [dst, jnp.zeros((pad,), jnp.int32)]).reshape(
            _NC, _NS, 2, _HALF, _CHUNK)
    z = jnp.zeros((_CHUNK, _D), jnp.float32)

    h = _dense_in(x, W_init)
    p = _sc_agg(h, src_p, dst_p, z)
    h = _dense_mid(p, W1, b1.reshape(1, _D))
    p = _sc_agg(h, src_p, dst_p, z)
    return _dense_fin(p, W2, b2.reshape(1, _D))


# speedup vs baseline: 1.0026x; 1.0026x over previous
import functools

import jax
import jax.numpy as jnp
from jax import lax
from jax.experimental import pallas as pl
from jax.experimental.pallas import tpu as pltpu
from jax.experimental.pallas import tpu_sc as plsc

_N = 10000
_D = 128
_E = 320000
_EPS = 1e-5

_NC = 2
_NS = 16
_CHUNK = 128
_NBUF = 2
_CH = 80
_EPT = _CH * _CHUNK
_EPAD = _NC * _NS * _EPT
_HALF = _CH // 2
_HPAD = 10016
_NPAD = 10112
_ZRPT = _NPAD // _NS


def _bn_relu(y):
    mean = jnp.mean(y, axis=0, keepdims=True)
    cen = y - mean
    var = jnp.mean(cen * cen, axis=0, keepdims=True)
    return jnp.maximum(cen * lax.rsqrt(var + _EPS), 0.0)


def _fc_in_body(x_ref, w_ref, o_ref):
    y = jnp.dot(x_ref[...], w_ref[...], preferred_element_type=jnp.float32)
    o_ref[pl.ds(0, _N), :] = _bn_relu(y)
    o_ref[pl.ds(_N, _HPAD - _N), :] = jnp.zeros((_HPAD - _N, _D), jnp.float32)


def _fc_mid_body(p_ref, w_ref, b_ref, o_ref):
    a = p_ref[0, pl.ds(0, _N), :] + p_ref[1, pl.ds(0, _N), :]
    y = jnp.dot(a, w_ref[...], preferred_element_type=jnp.float32)
    o_ref[pl.ds(0, _N), :] = _bn_relu(y + b_ref[...])
    o_ref[pl.ds(_N, _HPAD - _N), :] = jnp.zeros((_HPAD - _N, _D), jnp.float32)


def _fc_fin_body(p_ref, w_ref, b_ref, o_ref):
    a = p_ref[0, pl.ds(0, _N), :] + p_ref[1, pl.ds(0, _N), :]
    y = jnp.dot(a, w_ref[...], preferred_element_type=jnp.float32)
    o_ref[...] = _bn_relu(y + b_ref[...])


_dense_in = pl.pallas_call(
    _fc_in_body, out_shape=jax.ShapeDtypeStruct((_HPAD, _D), jnp.float32))
_dense_mid = pl.pallas_call(
    _fc_mid_body, out_shape=jax.ShapeDtypeStruct((_HPAD, _D), jnp.float32))
_dense_fin = pl.pallas_call(
    _fc_fin_body, out_shape=jax.ShapeDtypeStruct((_N, _D), jnp.float32))


@functools.partial(
    pl.kernel,
    mesh=plsc.VectorSubcoreMesh(core_axis_name="c", subcore_axis_name="s"),
    out_type=jax.ShapeDtypeStruct((_NC, _NPAD, _D), jnp.float32),
    scratch_types=[
        pltpu.VMEM((_HALF, _CHUNK), jnp.int32),
        pltpu.VMEM((_HALF, _CHUNK), jnp.int32),
        *[pltpu.VMEM((_CHUNK, _D), jnp.float32) for _ in range(_NBUF)],
        pltpu.VMEM_SHARED((_NPAD, _D), jnp.float32),
        *[pltpu.SemaphoreType.DMA for _ in range(_NBUF)],
    ],
)
def _sc_agg(h_hbm, src_hbm, dst_hbm, z_hbm, out_hbm,
            src_v, dst_v, r0, r1, agg_sh, s0, s1):
    rows = (r0, r1)
    sems = (s0, s1)
    c = lax.axis_index("c")
    s = lax.axis_index("s")
    with jax.named_scope("zero_phase"):
        pltpu.sync_copy(z_hbm, rows[0])
        for k in range(4):
            pltpu.sync_copy(rows[0],
                            agg_sh.at[pl.ds(s * _ZRPT + k * _CHUNK, _CHUNK)])
        pltpu.sync_copy(rows[0].at[pl.ds(0, _ZRPT - 4 * _CHUNK)],
                        agg_sh.at[pl.ds(s * _ZRPT + 4 * _CHUNK,
                                        _ZRPT - 4 * _CHUNK)])
        plsc.subcore_barrier()
    for half in range(2):
        with jax.named_scope(f"edge_half{half}"):
            pltpu.sync_copy(src_hbm.at[c, s, half], src_v)
            pltpu.sync_copy(dst_hbm.at[c, s, half], dst_v)
            for b in range(_NBUF):
                pltpu.async_copy(h_hbm.at[src_v.at[b]], rows[b], sems[b])

            def body(i, carry):
                for b in range(_NBUF):
                    j = i * _NBUF + b
                    pltpu.make_async_copy(h_hbm.at[src_v.at[j]], rows[b],
                                          sems[b]).wait()
                    pltpu.sync_copy(rows[b], agg_sh.at[dst_v.at[j]], add=True)
                    nxt = j + _NBUF

                    @pl.when(nxt < _HALF)
                    def _():
                        pltpu.async_copy(h_hbm.at[src_v.at[nxt]], rows[b],
                                         sems[b])

                return carry

            lax.fori_loop(0, _HALF // _NBUF, body, 0)
    with jax.named_scope("writeout"):
        plsc.subcore_barrier()
        last = {}
        for k in range(5):
            b = k & 1
            if k >= 2:
                pltpu.make_async_copy(*last[b]).wait()
            nrows = _CHUNK if k < 4 else _ZRPT - 4 * _CHUNK
            off = s * _ZRPT + k * _CHUNK
            stage = rows[b] if nrows == _CHUNK else rows[b].at[pl.ds(0, nrows)]
            pltpu.sync_copy(agg_sh.at[pl.ds(off, nrows)], stage)
            pltpu.async_copy(stage, out_hbm.at[c, pl.ds(off, nrows)], sems[b])
            last[b] = (stage, out_hbm.at[c, pl.ds(off, nrows)], sems[b])
        for b in (0, 1):
            pltpu.make_async_copy(*last[b]).wait()


def kernel(x, edge_index, W_init, W1, b1, W2, b2):
    src = edge_index[0]
    dst = edge_index[1]
    pad = _EPAD - _E
    src_p = jnp.concatenate(
        [src, jnp.full((pad,), _N, jnp.int32)]).reshape(
            _NC, _NS, 2, _HALF, _CHUNK)
    dst_p = jnp.concatenate(
        [dst, jnp.zeros((pad,), jnp.int32)]).reshape(
            _NC, _NS, 2, _HALF, _CHUNK)
    z = jnp.zeros((_CHUNK, _D), jnp.float32)

    h = _dense_in(x, W_init)
    p = _sc_agg(h, src_p, dst_p, z)
    h = _dense_mid(p, W1, b1.reshape(1, _D))
    p = _sc_agg(h, src_p, dst_p, z)
    return _dense_fin(p, W2, b2.reshape(1, _D))


# speedup vs baseline: 3.8112x; 3.8012x over previous
import functools

import jax
import jax.numpy as jnp
from jax import lax
from jax.experimental import pallas as pl
from jax.experimental.pallas import tpu as pltpu
from jax.experimental.pallas import tpu_sc as plsc

_N = 10000
_D = 128
_E = 320000
_EPS = 1e-5

_NC = 2
_NS = 16
_CHUNK = 128
_NBUF = 2
_CH = 80
_EPT = _CH * _CHUNK
_EPAD = _NC * _NS * _EPT
_HALF = _CH // 2
_HPAD = 10016
_NPAD = 10112
_ZRPT = _NPAD // _NS


def _bn_relu(y):
    mean = jnp.mean(y, axis=0, keepdims=True)
    cen = y - mean
    var = jnp.mean(cen * cen, axis=0, keepdims=True)
    return jnp.maximum(cen * lax.rsqrt(var + _EPS), 0.0)


def _fc_in_body(x_ref, w_ref, o_ref):
    y = jnp.dot(x_ref[...], w_ref[...], preferred_element_type=jnp.float32)
    o_ref[pl.ds(0, _N), :] = _bn_relu(y)
    o_ref[pl.ds(_N, _HPAD - _N), :] = jnp.zeros((_HPAD - _N, _D), jnp.float32)


def _fc_mid_body(p_ref, w_ref, b_ref, o_ref):
    a = p_ref[0, pl.ds(0, _N), :] + p_ref[1, pl.ds(0, _N), :]
    y = jnp.dot(a, w_ref[...], preferred_element_type=jnp.float32)
    o_ref[pl.ds(0, _N), :] = _bn_relu(y + b_ref[...])
    o_ref[pl.ds(_N, _HPAD - _N), :] = jnp.zeros((_HPAD - _N, _D), jnp.float32)


def _fc_fin_body(p_ref, w_ref, b_ref, o_ref):
    a = p_ref[0, pl.ds(0, _N), :] + p_ref[1, pl.ds(0, _N), :]
    y = jnp.dot(a, w_ref[...], preferred_element_type=jnp.float32)
    o_ref[...] = _bn_relu(y + b_ref[...])


_dense_in = pl.pallas_call(
    _fc_in_body, out_shape=jax.ShapeDtypeStruct((_HPAD, _D), jnp.float32))
_dense_mid = pl.pallas_call(
    _fc_mid_body, out_shape=jax.ShapeDtypeStruct((_HPAD, _D), jnp.float32))
_dense_fin = pl.pallas_call(
    _fc_fin_body, out_shape=jax.ShapeDtypeStruct((_N, _D), jnp.float32))


@functools.partial(
    pl.kernel,
    mesh=plsc.VectorSubcoreMesh(core_axis_name="c", subcore_axis_name="s"),
    out_type=jax.ShapeDtypeStruct((_NC, _NPAD, _D), jnp.float32),
    scratch_types=[
        pltpu.VMEM((_HALF, _CHUNK), jnp.int32),
        pltpu.VMEM((_HALF, _CHUNK), jnp.int32),
        *[pltpu.VMEM((_CHUNK, _D), jnp.float32) for _ in range(_NBUF)],
        pltpu.VMEM_SHARED((_NPAD, _D), jnp.float32),
        *[pltpu.SemaphoreType.DMA for _ in range(_NBUF)],
    ],
)
def _sc_agg(h_hbm, src_hbm, dst_hbm, z_hbm, out_hbm,
            src_v, dst_v, r0, r1, agg_sh, s0, s1):
    rows = (r0, r1)
    sems = (s0, s1)
    c = lax.axis_index("c")
    s = lax.axis_index("s")
    with jax.named_scope("zero_phase"):
        pltpu.sync_copy(z_hbm, rows[0])
        for k in range(4):
            pltpu.sync_copy(rows[0],
                            agg_sh.at[pl.ds(s * _ZRPT + k * _CHUNK, _CHUNK)])
        pltpu.sync_copy(rows[0].at[pl.ds(0, _ZRPT - 4 * _CHUNK)],
                        agg_sh.at[pl.ds(s * _ZRPT + 4 * _CHUNK,
                                        _ZRPT - 4 * _CHUNK)])
        plsc.subcore_barrier()
    for half in range(2):
        with jax.named_scope(f"edge_half{half}"):
            pltpu.sync_copy(src_hbm.at[c, s, half], src_v)
            pltpu.sync_copy(dst_hbm.at[c, s, half], dst_v)
            for b in range(_NBUF):
                pltpu.async_copy(h_hbm.at[src_v.at[b]], rows[b], sems[b])

            def body(i, carry):
                for b in range(_NBUF):
                    j = i * _NBUF + b
                    pltpu.make_async_copy(h_hbm.at[src_v.at[j]], rows[b],
                                          sems[b]).wait()
                    pltpu.sync_copy(rows[b], agg_sh.at[dst_v.at[j]], add=True)
                    nxt = j + _NBUF

                    @pl.when(nxt < _HALF)
                    def _():
                        pltpu.async_copy(h_hbm.at[src_v.at[nxt]], rows[b],
                                         sems[b])

                return carry

            lax.fori_loop(0, _HALF // _NBUF, body, 0)
    with jax.named_scope("writeout"):
        plsc.subcore_barrier()
        last = {}
        for k in range(5):
            b = k & 1
            if k >= 2:
                pltpu.make_async_copy(*last[b]).wait()
            nrows = _CHUNK if k < 4 else _ZRPT - 4 * _CHUNK
            off = s * _ZRPT + k * _CHUNK
            stage = rows[b] if nrows == _CHUNK else rows[b].at[pl.ds(0, nrows)]
            pltpu.sync_copy(agg_sh.at[pl.ds(off, nrows)], stage)
            pltpu.async_copy(stage, out_hbm.at[c, pl.ds(off, nrows)], sems[b])
            last[b] = (stage, out_hbm.at[c, pl.ds(off, nrows)], sems[b])
        for b in (0, 1):
            pltpu.make_async_copy(*last[b]).wait()


def kernel(x, edge_index, W_init, W1, b1, W2, b2):
    src = edge_index[0]
    dst = edge_index[1]
    pad = _EPAD - _E
    pad_iota = jnp.arange(pad, dtype=jnp.int32)
    src_p = jnp.concatenate(
        [src, _N + (pad_iota % (_HPAD - _N))]).reshape(
            _NC, _NS, 2, _HALF, _CHUNK)
    dst_p = jnp.concatenate(
        [dst, pad_iota % _N]).reshape(
            _NC, _NS, 2, _HALF, _CHUNK)
    z = jnp.zeros((_CHUNK, _D), jnp.float32)

    h = _dense_in(x, W_init)
    p = _sc_agg(h, src_p, dst_p, z)
    h = _dense_mid(p, W1, b1.reshape(1, _D))
    p = _sc_agg(h, src_p, dst_p, z)
    return _dense_fin(p, W2, b2.reshape(1, _D))


# speedup vs baseline: 3.8633x; 1.0137x over previous
import functools

import jax
import jax.numpy as jnp
from jax import lax
from jax.experimental import pallas as pl
from jax.experimental.pallas import tpu as pltpu
from jax.experimental.pallas import tpu_sc as plsc

_N = 10000
_D = 128
_E = 320000
_EPS = 1e-5

_NC = 2
_NS = 16
_CHUNK = 128
_NBUF = 2
_CH = 80
_EPT = _CH * _CHUNK
_EPAD = _NC * _NS * _EPT
_HALF = _CH // 2
_HPAD = 10016
_NPAD = 10112
_ZRPT = _NPAD // _NS
_NT = _NC * _NS
_EMAIN = (_NT - 1) * _EPT


def _bn_relu(y):
    mean = jnp.mean(y, axis=0, keepdims=True)
    cen = y - mean
    var = jnp.mean(cen * cen, axis=0, keepdims=True)
    return jnp.maximum(cen * lax.rsqrt(var + _EPS), 0.0)


def _fc_in_body(x_ref, w_ref, o_ref):
    y = jnp.dot(x_ref[...], w_ref[...], preferred_element_type=jnp.float32)
    o_ref[pl.ds(0, _N), :] = _bn_relu(y)
    o_ref[pl.ds(_N, _HPAD - _N), :] = jnp.zeros((_HPAD - _N, _D), jnp.float32)


def _fc_mid_body(p_ref, w_ref, b_ref, o_ref):
    a = p_ref[0, pl.ds(0, _N), :] + p_ref[1, pl.ds(0, _N), :]
    y = jnp.dot(a, w_ref[...], preferred_element_type=jnp.float32)
    o_ref[pl.ds(0, _N), :] = _bn_relu(y + b_ref[...])
    o_ref[pl.ds(_N, _HPAD - _N), :] = jnp.zeros((_HPAD - _N, _D), jnp.float32)


def _fc_fin_body(p_ref, w_ref, b_ref, o_ref):
    a = p_ref[0, pl.ds(0, _N), :] + p_ref[1, pl.ds(0, _N), :]
    y = jnp.dot(a, w_ref[...], preferred_element_type=jnp.float32)
    o_ref[...] = _bn_relu(y + b_ref[...])


_dense_in = pl.pallas_call(
    _fc_in_body, out_shape=jax.ShapeDtypeStruct((_HPAD, _D), jnp.float32))
_dense_mid = pl.pallas_call(
    _fc_mid_body, out_shape=jax.ShapeDtypeStruct((_HPAD, _D), jnp.float32))
_dense_fin = pl.pallas_call(
    _fc_fin_body, out_shape=jax.ShapeDtypeStruct((_N, _D), jnp.float32))


@functools.partial(
    pl.kernel,
    mesh=plsc.VectorSubcoreMesh(core_axis_name="c", subcore_axis_name="s"),
    out_type=jax.ShapeDtypeStruct((_NC, _NPAD, _D), jnp.float32),
    scratch_types=[
        pltpu.VMEM((_HALF, _CHUNK), jnp.int32),
        pltpu.VMEM((_HALF, _CHUNK), jnp.int32),
        *[pltpu.VMEM((_CHUNK, _D), jnp.float32) for _ in range(_NBUF)],
        pltpu.VMEM_SHARED((_NPAD, _D), jnp.float32),
        *[pltpu.SemaphoreType.DMA for _ in range(_NBUF)],
    ],
)
def _sc_agg(h_hbm, src_hbm, dst_hbm, srct_hbm, dstt_hbm, z_hbm, out_hbm,
            src_v, dst_v, r0, r1, agg_sh, s0, s1):
    rows = (r0, r1)
    sems = (s0, s1)
    c = lax.axis_index("c")
    s = lax.axis_index("s")
    t = c * _NS + s
    with jax.named_scope("zero_phase"):
        pltpu.sync_copy(z_hbm, rows[0])
        for k in range(4):
            pltpu.sync_copy(rows[0],
                            agg_sh.at[pl.ds(s * _ZRPT + k * _CHUNK, _CHUNK)])
        pltpu.sync_copy(rows[0].at[pl.ds(0, _ZRPT - 4 * _CHUNK)],
                        agg_sh.at[pl.ds(s * _ZRPT + 4 * _CHUNK,
                                        _ZRPT - 4 * _CHUNK)])
        plsc.subcore_barrier()
    for half in range(2):
        with jax.named_scope(f"edge_half{half}"):
            @pl.when(t < _NT - 1)
            def _():
                pltpu.sync_copy(src_hbm.at[t, half], src_v)
                pltpu.sync_copy(dst_hbm.at[t, half], dst_v)

            @pl.when(t == _NT - 1)
            def _():
                pltpu.sync_copy(srct_hbm.at[half], src_v)
                pltpu.sync_copy(dstt_hbm.at[half], dst_v)
            for b in range(_NBUF):
                pltpu.async_copy(h_hbm.at[src_v.at[b]], rows[b], sems[b])

            def body(i, carry):
                for b in range(_NBUF):
                    j = i * _NBUF + b
                    pltpu.make_async_copy(h_hbm.at[src_v.at[j]], rows[b],
                                          sems[b]).wait()
                    pltpu.sync_copy(rows[b], agg_sh.at[dst_v.at[j]], add=True)
                    nxt = j + _NBUF

                    @pl.when(nxt < _HALF)
                    def _():
                        pltpu.async_copy(h_hbm.at[src_v.at[nxt]], rows[b],
                                         sems[b])

                return carry

            lax.fori_loop(0, _HALF // _NBUF, body, 0)
    with jax.named_scope("writeout"):
        plsc.subcore_barrier()
        last = {}
        for k in range(5):
            b = k & 1
            if k >= 2:
                pltpu.make_async_copy(*last[b]).wait()
            nrows = _CHUNK if k < 4 else _ZRPT - 4 * _CHUNK
            off = s * _ZRPT + k * _CHUNK
            stage = rows[b] if nrows == _CHUNK else rows[b].at[pl.ds(0, nrows)]
            pltpu.sync_copy(agg_sh.at[pl.ds(off, nrows)], stage)
            pltpu.async_copy(stage, out_hbm.at[c, pl.ds(off, nrows)], sems[b])
            last[b] = (stage, out_hbm.at[c, pl.ds(off, nrows)], sems[b])
        for b in (0, 1):
            pltpu.make_async_copy(*last[b]).wait()


def kernel(x, edge_index, W_init, W1, b1, W2, b2):
    pad = _EPAD - _E
    src_main = edge_index[0, :_EMAIN].reshape(_NT - 1, 2, _HALF, _CHUNK)
    dst_main = edge_index[1, :_EMAIN].reshape(_NT - 1, 2, _HALF, _CHUNK)
    pad_iota = jnp.arange(pad, dtype=jnp.int32)
    src_tail = jnp.concatenate(
        [edge_index[0, _EMAIN:], _N + (pad_iota % (_HPAD - _N))]).reshape(
            2, _HALF, _CHUNK)
    dst_tail = jnp.concatenate(
        [edge_index[1, _EMAIN:], pad_iota % _N]).reshape(2, _HALF, _CHUNK)
    z = jnp.zeros((_CHUNK, _D), jnp.float32)

    h = _dense_in(x, W_init)
    p = _sc_agg(h, src_main, dst_main, src_tail, dst_tail, z)
    h = _dense_mid(p, W1, b1.reshape(1, _D))
    p = _sc_agg(h, src_main, dst_main, src_tail, dst_tail, z)
    return _dense_fin(p, W2, b2.reshape(1, _D))


# speedup vs baseline: 4.0183x; 1.0401x over previous
import functools

import jax
import jax.numpy as jnp
from jax import lax
from jax.experimental import pallas as pl
from jax.experimental.pallas import tpu as pltpu
from jax.experimental.pallas import tpu_sc as plsc

_N = 10000
_D = 128
_E = 320000
_EPS = 1e-5

_NC = 2
_NS = 16
_CHUNK = 128
_NBUF = 2
_CH = 80
_EPT = _CH * _CHUNK
_EPAD = _NC * _NS * _EPT
_HALF = _CH // 2
_HPAD = 10016
_NPAD = 10112
_ZRPT = _NPAD // _NS
_NT = _NC * _NS
_EMAIN = (_NT - 1) * _EPT


def _bn_relu(y):
    mean = jnp.mean(y, axis=0, keepdims=True)
    cen = y - mean
    var = jnp.mean(cen * cen, axis=0, keepdims=True)
    return jnp.maximum(cen * lax.rsqrt(var + _EPS), 0.0)


def _fc_in_body(x_ref, w_ref, o_ref):
    y = jnp.dot(x_ref[...], w_ref[...], preferred_element_type=jnp.float32)
    o_ref[pl.ds(0, _N), :] = _bn_relu(y)
    o_ref[pl.ds(_N, _HPAD - _N), :] = jnp.zeros((_HPAD - _N, _D), jnp.float32)


def _fc_mid_body(p_ref, w_ref, b_ref, o_ref):
    a = p_ref[0, pl.ds(0, _N), :] + p_ref[1, pl.ds(0, _N), :]
    y = jnp.dot(a, w_ref[...], preferred_element_type=jnp.float32)
    o_ref[pl.ds(0, _N), :] = _bn_relu(y + b_ref[...])
    o_ref[pl.ds(_N, _HPAD - _N), :] = jnp.zeros((_HPAD - _N, _D), jnp.float32)


def _fc_fin_body(p_ref, w_ref, b_ref, o_ref):
    a = p_ref[0, pl.ds(0, _N), :] + p_ref[1, pl.ds(0, _N), :]
    y = jnp.dot(a, w_ref[...], preferred_element_type=jnp.float32)
    o_ref[...] = _bn_relu(y + b_ref[...])


_dense_in = pl.pallas_call(
    _fc_in_body, out_shape=jax.ShapeDtypeStruct((_HPAD, _D), jnp.float32))
_dense_mid = pl.pallas_call(
    _fc_mid_body, out_shape=jax.ShapeDtypeStruct((_HPAD, _D), jnp.float32))
_dense_fin = pl.pallas_call(
    _fc_fin_body, out_shape=jax.ShapeDtypeStruct((_N, _D), jnp.float32))


@functools.partial(
    pl.kernel,
    mesh=plsc.VectorSubcoreMesh(core_axis_name="c", subcore_axis_name="s"),
    out_type=jax.ShapeDtypeStruct((_NC, _NPAD, _D), jnp.float32),
    scratch_types=[
        pltpu.VMEM((_HALF, _CHUNK), jnp.int32),
        pltpu.VMEM((_HALF, _CHUNK), jnp.int32),
        *[pltpu.VMEM((_CHUNK, _D), jnp.float32) for _ in range(_NBUF)],
        pltpu.VMEM_SHARED((_NPAD, _D), jnp.float32),
        *[pltpu.SemaphoreType.DMA for _ in range(_NBUF)],
    ],
)
def _sc_agg(h_hbm, em_hbm, srct_hbm, dstt_hbm, out_hbm,
            src_v, dst_v, r0, r1, agg_sh, s0, s1):
    rows = (r0, r1)
    sems = (s0, s1)
    c = lax.axis_index("c")
    s = lax.axis_index("s")
    t = c * _NS + s
    with jax.named_scope("zero_phase"):
        z16 = jnp.zeros((16,), jnp.float32)

        def zbody(r, carry):
            for k8 in range(_D // 16):
                rows[0][r, pl.ds(k8 * 16, 16)] = z16
            return carry

        lax.fori_loop(0, _CHUNK, zbody, 0)
        for k in range(4):
            pltpu.sync_copy(rows[0],
                            agg_sh.at[pl.ds(s * _ZRPT + k * _CHUNK, _CHUNK)])
        pltpu.sync_copy(rows[0].at[pl.ds(0, _ZRPT - 4 * _CHUNK)],
                        agg_sh.at[pl.ds(s * _ZRPT + 4 * _CHUNK,
                                        _ZRPT - 4 * _CHUNK)])
        plsc.subcore_barrier()
    for half in range(2):
        with jax.named_scope(f"edge_half{half}"):
            @pl.when(t < _NT - 1)
            def _():
                base = t * _CH + half * _HALF
                pltpu.sync_copy(em_hbm.at[0, pl.ds(base, _HALF)], src_v)
                pltpu.sync_copy(em_hbm.at[1, pl.ds(base, _HALF)], dst_v)

            @pl.when(t == _NT - 1)
            def _():
                pltpu.sync_copy(srct_hbm.at[half], src_v)
                pltpu.sync_copy(dstt_hbm.at[half], dst_v)
            for b in range(_NBUF):
                pltpu.async_copy(h_hbm.at[src_v.at[b]], rows[b], sems[b])

            def body(i, carry):
                for b in range(_NBUF):
                    j = i * _NBUF + b
                    pltpu.make_async_copy(h_hbm.at[src_v.at[j]], rows[b],
                                          sems[b]).wait()
                    pltpu.sync_copy(rows[b], agg_sh.at[dst_v.at[j]], add=True)
                    nxt = j + _NBUF

                    @pl.when(nxt < _HALF)
                    def _():
                        pltpu.async_copy(h_hbm.at[src_v.at[nxt]], rows[b],
                                         sems[b])

                return carry

            lax.fori_loop(0, _HALF // _NBUF, body, 0)
    with jax.named_scope("writeout"):
        plsc.subcore_barrier()
        last = {}
        for k in range(5):
            b = k & 1
            if k >= 2:
                pltpu.make_async_copy(*last[b]).wait()
            nrows = _CHUNK if k < 4 else _ZRPT - 4 * _CHUNK
            off = s * _ZRPT + k * _CHUNK
            stage = rows[b] if nrows == _CHUNK else rows[b].at[pl.ds(0, nrows)]
            pltpu.sync_copy(agg_sh.at[pl.ds(off, nrows)], stage)
            pltpu.async_copy(stage, out_hbm.at[c, pl.ds(off, nrows)], sems[b])
            last[b] = (stage, out_hbm.at[c, pl.ds(off, nrows)], sems[b])
        for b in (0, 1):
            pltpu.make_async_copy(*last[b]).wait()


def kernel(x, edge_index, W_init, W1, b1, W2, b2):
    pad = _EPAD - _E
    em = edge_index.reshape(2, _E // _CHUNK, _CHUNK)
    pad_iota = jnp.arange(pad, dtype=jnp.int32)
    src_tail = jnp.concatenate(
        [edge_index[0, _EMAIN:], _N + (pad_iota % (_HPAD - _N))]).reshape(
            2, _HALF, _CHUNK)
    dst_tail = jnp.concatenate(
        [edge_index[1, _EMAIN:], pad_iota % _N]).reshape(2, _HALF, _CHUNK)

    h = _dense_in(x, W_init)
    p = _sc_agg(h, em, src_tail, dst_tail)
    h = _dense_mid(p, W1, b1.reshape(1, _D))
    p = _sc_agg(h, em, src_tail, dst_tail)
    return _dense_fin(p, W2, b2.reshape(1, _D))
